# Initial kernel scaffold; baseline (speedup 1.0000x reference)
import functools

import jax
import jax.numpy as jnp
from jax import lax
from jax.experimental import pallas as pl
from jax.experimental.pallas import tpu as pltpu, tpu_sc as plsc

_LANES = 16
_NUM_WORKERS = 32
_NUM_CORES = 2


def _sc_body(energy_hbm, types_hbm, scales_hbm, shifts_hbm, out_hbm,
             energy_v, types_v, out_v, scales_v, shifts_v):
    wid = lax.axis_index("s") * _NUM_CORES + lax.axis_index("c")
    per_w = energy_hbm.shape[0] // _NUM_WORKERS
    base = wid * per_w
    pltpu.sync_copy(types_hbm.at[pl.ds(base, per_w)], types_v)
    pltpu.sync_copy(energy_hbm.at[pl.ds(base, per_w)], energy_v)
    pltpu.sync_copy(scales_hbm, scales_v)
    pltpu.sync_copy(shifts_hbm, shifts_v)

    def step(i, carry):
        sl = pl.ds(i * _LANES, _LANES)
        t = types_v[sl]
        s = plsc.load_gather(scales_v, [t])
        b = plsc.load_gather(shifts_v, [t])
        out_v[sl] = b + s * energy_v[sl]
        return carry

    lax.fori_loop(0, per_w // _LANES, step, 0)
    pltpu.sync_copy(out_v, out_hbm.at[pl.ds(base, per_w)])


def _make_sc_call(n_pad):
    per_w = n_pad // _NUM_WORKERS
    mesh = plsc.VectorSubcoreMesh(core_axis_name="c", subcore_axis_name="s")
    return pl.kernel(
        _sc_body,
        out_type=jax.ShapeDtypeStruct((n_pad,), jnp.float32),
        mesh=mesh,
        scratch_types=[
            pltpu.VMEM((per_w,), jnp.float32),
            pltpu.VMEM((per_w,), jnp.int32),
            pltpu.VMEM((per_w,), jnp.float32),
            pltpu.VMEM((_LANES * 4,), jnp.float32),
            pltpu.VMEM((_LANES * 4,), jnp.float32),
        ],
    )


def kernel(atomic_energy, atom_types, scales, shifts):
    n = atomic_energy.shape[0]
    x = atomic_energy.reshape(-1)
    t = atom_types.reshape(-1).astype(jnp.int32)
    chunk = _LANES * _NUM_WORKERS
    n_pad = ((n + chunk - 1) // chunk) * chunk
    if n_pad != n:
        x = jnp.pad(x, (0, n_pad - n))
        t = jnp.pad(t, (0, n_pad - n))
    out = _make_sc_call(n_pad)(x, t, scales, shifts)
    return out[:n].reshape(-1, 1)


# speedup vs baseline: 3.6208x; 3.6208x over previous
import functools

import jax
import jax.numpy as jnp
from jax import lax
from jax.experimental import pallas as pl
from jax.experimental.pallas import tpu as pltpu, tpu_sc as plsc

_LANES = 16
_NUM_WORKERS = 32
_NUM_CORES = 2


def _sc_body(energy_hbm, types_hbm, scales_hbm, shifts_hbm, out_hbm,
             energy_v, types_v, out_v, scales_v, shifts_v):
    wid = lax.axis_index("s") * _NUM_CORES + lax.axis_index("c")
    per_w = energy_hbm.shape[0] // _NUM_WORKERS
    base = wid * per_w
    pltpu.sync_copy(types_hbm.at[pl.ds(base, per_w)], types_v)
    pltpu.sync_copy(energy_hbm.at[pl.ds(base, per_w)], energy_v)
    pltpu.sync_copy(scales_hbm, scales_v)
    pltpu.sync_copy(shifts_hbm, shifts_v)

    def step(i, carry):
        sl = pl.ds(i * _LANES, _LANES)
        t = types_v[sl]
        s = plsc.load_gather(scales_v, [t])
        b = plsc.load_gather(shifts_v, [t])
        out_v[sl] = b + s * energy_v[sl]
        return carry

    lax.fori_loop(0, per_w // _LANES, step, 0)
    pltpu.sync_copy(out_v, out_hbm.at[pl.ds(base, per_w)])


def _make_sc_call(n_pad):
    per_w = n_pad // _NUM_WORKERS
    mesh = plsc.VectorSubcoreMesh(core_axis_name="c", subcore_axis_name="s")
    return pl.kernel(
        _sc_body,
        out_type=jax.ShapeDtypeStruct((n_pad,), jnp.float32),
        mesh=mesh,
        scratch_types=[
            pltpu.VMEM((per_w,), jnp.float32),
            pltpu.VMEM((per_w,), jnp.int32),
            pltpu.VMEM((per_w,), jnp.float32),
            pltpu.VMEM((_LANES * 4,), jnp.float32),
            pltpu.VMEM((_LANES * 4,), jnp.float32),
        ],
        compiler_params=pltpu.CompilerParams(needs_layout_passes=False),
    )


def kernel(atomic_energy, atom_types, scales, shifts):
    n = atomic_energy.shape[0]
    x = atomic_energy.reshape(-1)
    t = atom_types.reshape(-1).astype(jnp.int32)
    chunk = _LANES * _NUM_WORKERS
    n_pad = ((n + chunk - 1) // chunk) * chunk
    if n_pad != n:
        x = jnp.pad(x, (0, n_pad - n))
        t = jnp.pad(t, (0, n_pad - n))
    out = _make_sc_call(n_pad)(x, t, scales, shifts)
    return out[:n].reshape(-1, 1)
